# baseline (speedup 1.0000x reference)
import functools

import jax
import jax.numpy as jnp
from jax import lax
from jax.experimental import pallas as pl
from jax.experimental.pallas import tpu as pltpu
from jax.experimental.pallas import tpu_sc as plsc

VEC = 64
ROWS = 4096 * 200
NC, NS = 2, 16
NW = NC * NS
RPW = ROWS // NW
IMINOR = 128
IROWS_PW = RPW // IMINOR
CH = 4
CHUNK = CH * IMINOR
NIT = IROWS_PW // CH


def _emb_body(inds_hbm, w_hbm, out_hbm,
              idx_v, rows0, rows1, gsem0, gsem1, wsem0, wsem1):
    wid = lax.axis_index("s") * NC + lax.axis_index("c")
    irow0 = wid * IROWS_PW
    orow0 = wid * RPW

    rows = (rows0, rows1)
    gsem = (gsem0, gsem1)
    wsem = (wsem0, wsem1)

    pltpu.sync_copy(inds_hbm.at[pl.ds(irow0, IROWS_PW)], idx_v)

    def fire_gathers(it, b):
        for k in range(CH):
            pltpu.async_copy(
                w_hbm.at[idx_v.at[it * CH + k]],
                rows[b].at[pl.ds(k * IMINOR, IMINOR)],
                gsem[b],
            )

    def wait_gathers(b):
        pltpu.make_async_copy(w_hbm.at[pl.ds(0, CHUNK)], rows[b], gsem[b]).wait()

    def writeback(it, b):
        return pltpu.async_copy(
            rows[b], out_hbm.at[pl.ds(orow0 + it * CHUNK, CHUNK)], wsem[b])

    def wait_writeback(it, b):
        pltpu.make_async_copy(
            rows[b], out_hbm.at[pl.ds(orow0 + it * CHUNK, CHUNK)], wsem[b]).wait()

    fire_gathers(0, 0)

    @pl.loop(0, NIT, step=2)
    def _steps(g):
        for b in (0, 1):
            it = g + b
            nb = 1 - b

            @pl.when(it + 1 < NIT)
            def _():
                @pl.when(it >= 1)
                def _():
                    wait_writeback(it - 1, nb)
                fire_gathers(it + 1, nb)

            wait_gathers(b)
            writeback(it, b)

    wait_writeback(NIT - 2, 0)
    wait_writeback(NIT - 1, 1)


_emb = functools.partial(
    pl.kernel,
    out_type=jax.ShapeDtypeStruct((ROWS, VEC), jnp.float32),
    mesh=plsc.VectorSubcoreMesh(core_axis_name="c", subcore_axis_name="s"),
    scratch_types=[
        pltpu.VMEM((IROWS_PW, IMINOR), jnp.int32),
        pltpu.VMEM((CHUNK, VEC), jnp.float32),
        pltpu.VMEM((CHUNK, VEC), jnp.float32),
        pltpu.SemaphoreType.DMA,
        pltpu.SemaphoreType.DMA,
        pltpu.SemaphoreType.DMA,
        pltpu.SemaphoreType.DMA,
    ],
    compiler_params=pltpu.CompilerParams(use_tc_tiling_on_sc=False),
)(_emb_body)


@jax.jit
def kernel(inds, W):
    b, h = inds.shape
    inds2d = inds.reshape(ROWS // IMINOR, IMINOR)
    out = _emb(inds2d, W)
    return out.reshape(b, h, VEC)


# speedup vs baseline: 1.2198x; 1.2198x over previous
import functools

import jax
import jax.numpy as jnp
from jax import lax
from jax.experimental import pallas as pl
from jax.experimental.pallas import tpu as pltpu
from jax.experimental.pallas import tpu_sc as plsc

VEC = 64
VECP = 128
BATCH = 4096
HIST = 200
NC, NS = 2, 16
NW = NC * NS
BPW = BATCH // NW
H0 = 128
H1 = HIST - H0


def _emb_body(inds_hbm, w_hbm, out_hbm,
              idx_v, rows0, rows1, gsem0, gsem1, wsem0, wsem1):
    wid = lax.axis_index("s") * NC + lax.axis_index("c")
    b0 = wid * BPW

    rows = (rows0, rows1)
    gsem = (gsem0, gsem1)
    wsem = (wsem0, wsem1)

    pltpu.sync_copy(inds_hbm.at[pl.ds(b0, BPW)], idx_v)

    def fire_gathers(bl, p):
        pltpu.async_copy(
            w_hbm.at[idx_v.at[bl, pl.ds(0, H0)]],
            rows[p].at[pl.ds(0, H0)], gsem[p])
        pltpu.async_copy(
            w_hbm.at[idx_v.at[bl, pl.ds(H0, H1)]],
            rows[p].at[pl.ds(H0, H1)], gsem[p])

    def wait_gathers(p):
        pltpu.make_async_copy(w_hbm.at[pl.ds(0, HIST)], rows[p], gsem[p]).wait()

    def writeback(bl, p):
        pltpu.async_copy(
            rows[p], out_hbm.at[pl.ds((b0 + bl) * HIST, HIST)], wsem[p])

    def wait_writeback(bl, p):
        pltpu.make_async_copy(
            rows[p], out_hbm.at[pl.ds((b0 + bl) * HIST, HIST)], wsem[p]).wait()

    fire_gathers(0, 0)

    @pl.loop(0, BPW, step=2)
    def _steps(g):
        for p in (0, 1):
            bl = g + p
            np_ = 1 - p

            @pl.when(bl + 1 < BPW)
            def _():
                @pl.when(bl >= 1)
                def _():
                    wait_writeback(bl - 1, np_)
                fire_gathers(bl + 1, np_)

            wait_gathers(p)
            writeback(bl, p)

    wait_writeback(BPW - 2, 0)
    wait_writeback(BPW - 1, 1)


_emb = functools.partial(
    pl.kernel,
    out_type=jax.ShapeDtypeStruct((BATCH * HIST, VECP), jnp.float32),
    mesh=plsc.VectorSubcoreMesh(core_axis_name="c", subcore_axis_name="s"),
    scratch_types=[
        pltpu.VMEM((BPW, HIST), jnp.int32),
        pltpu.VMEM((HIST, VECP), jnp.float32),
        pltpu.VMEM((HIST, VECP), jnp.float32),
        pltpu.SemaphoreType.DMA,
        pltpu.SemaphoreType.DMA,
        pltpu.SemaphoreType.DMA,
        pltpu.SemaphoreType.DMA,
    ],
    compiler_params=pltpu.CompilerParams(use_tc_tiling_on_sc=True),
)(_emb_body)


@jax.jit
def kernel(inds, W):
    Wp = jnp.pad(W, ((0, 0), (0, VECP - VEC)))
    out = _emb(inds, Wp)
    return out[:, :VEC].reshape(BATCH, HIST, VEC)
